# Initial kernel scaffold; baseline (speedup 1.0000x reference)
import jax
import jax.numpy as jnp
from jax.experimental import pallas as pl


def kernel(z, x, edge_index):
    raise NotImplementedError("write your pallas kernel here")


# speedup vs baseline: 2.5521x; 2.5521x over previous
import functools

import jax
import jax.numpy as jnp
from jax import lax
from jax.experimental import pallas as pl
from jax.experimental.pallas import tpu as pltpu
from jax.experimental.pallas import tpu_sc as plsc

NC = 2
NS = 16
NW = NC * NS
LANES = 16

CHUNK = 40


def _make_sc_kernel(e: int, d: int):
    e_per_w = e // NW
    n_chunks = e_per_w // CHUNK
    mesh = plsc.VectorSubcoreMesh(core_axis_name="c", subcore_axis_name="s")

    @functools.partial(
        pl.kernel,
        mesh=mesh,
        out_type=jax.ShapeDtypeStruct((e, d), jnp.float32),
        scratch_types=[
            pltpu.VMEM((CHUNK,), jnp.int32),
            pltpu.VMEM((CHUNK,), jnp.int32),
            pltpu.VMEM((CHUNK, d), jnp.float32),
            pltpu.VMEM((CHUNK, d), jnp.float32),
            pltpu.SemaphoreType.DMA,
            pltpu.SemaphoreType.DMA,
        ],
    )
    def k(z_hbm, x_hbm, src_hbm, dst_hbm, out_hbm, idx_s, idx_d, zr, xr, s1, s2):
        wid = lax.axis_index("s") * NC + lax.axis_index("c")
        base = wid * e_per_w

        def chunk_body(j, carry):
            off = base + j * CHUNK
            pltpu.sync_copy(src_hbm.at[pl.ds(off, CHUNK)], idx_s)
            pltpu.sync_copy(dst_hbm.at[pl.ds(off, CHUNK)], idx_d)
            cz = pltpu.async_copy(z_hbm.at[idx_s], zr, s1)
            cx = pltpu.async_copy(x_hbm.at[idx_d], xr, s2)
            cz.wait()
            cx.wait()

            def row_body(r, c2):
                for cc in range(d // LANES):
                    sl = pl.ds(cc * LANES, LANES)
                    zr[r, sl] = zr[r, sl] * xr[r, sl]
                return c2

            lax.fori_loop(0, CHUNK, row_body, 0, unroll=False)
            pltpu.sync_copy(zr, out_hbm.at[pl.ds(off, CHUNK)])
            return carry

        lax.fori_loop(0, n_chunks, chunk_body, 0, unroll=False)

    return k


def kernel(z, x, edge_index):
    e = edge_index.shape[1]
    d = z.shape[1]
    src = edge_index[0].astype(jnp.int32)
    dst = edge_index[1].astype(jnp.int32)
    return _make_sc_kernel(e, d)(z, x, src, dst)


# speedup vs baseline: 4.0790x; 1.5983x over previous
import functools

import jax
import jax.numpy as jnp
from jax import lax
from jax.experimental import pallas as pl
from jax.experimental.pallas import tpu as pltpu
from jax.experimental.pallas import tpu_sc as plsc

NC = 2
NS = 16
NW = NC * NS
LANES = 16

CHUNK = 40


def _make_sc_kernel(e: int, d: int):
    e_per_w = e // NW
    n_chunks = e_per_w // CHUNK
    n_pairs = n_chunks // 2
    mesh = plsc.VectorSubcoreMesh(core_axis_name="c", subcore_axis_name="s")

    @functools.partial(
        pl.kernel,
        mesh=mesh,
        out_type=jax.ShapeDtypeStruct((e, d), jnp.float32),
        scratch_types=[
            pltpu.VMEM((2, CHUNK), jnp.int32),
            pltpu.VMEM((2, CHUNK), jnp.int32),
            pltpu.VMEM((2, CHUNK, d), jnp.float32),
            pltpu.VMEM((2, CHUNK, d), jnp.float32),
            pltpu.SemaphoreType.DMA,
            pltpu.SemaphoreType.DMA,
            pltpu.SemaphoreType.DMA,
            pltpu.SemaphoreType.DMA,
            pltpu.SemaphoreType.DMA,
            pltpu.SemaphoreType.DMA,
        ],
    )
    def k(z_hbm, x_hbm, src_hbm, dst_hbm, out_hbm, idx_s, idx_d, zr, xr,
          gz0, gx0, gz1, gx1, o0, o1):
        gz = (gz0, gz1)
        gx = (gx0, gx1)
        osem = (o0, o1)
        wid = lax.axis_index("s") * NC + lax.axis_index("c")
        base = wid * e_per_w

        def start_gather(off, b):
            pltpu.sync_copy(src_hbm.at[pl.ds(off, CHUNK)], idx_s.at[b])
            pltpu.sync_copy(dst_hbm.at[pl.ds(off, CHUNK)], idx_d.at[b])
            pltpu.async_copy(z_hbm.at[idx_s.at[b]], zr.at[b], gz[b])
            pltpu.async_copy(x_hbm.at[idx_d.at[b]], xr.at[b], gx[b])

        def wait_gather(b):
            pltpu.make_async_copy(z_hbm.at[idx_s.at[b]], zr.at[b], gz[b]).wait()
            pltpu.make_async_copy(x_hbm.at[idx_d.at[b]], xr.at[b], gx[b]).wait()

        def wait_write(b):
            pltpu.make_async_copy(
                zr.at[b], out_hbm.at[pl.ds(0, CHUNK)], osem[b]).wait()

        def compute(b):
            def row_body(r, c2):
                for cc in range(d // LANES):
                    sl = pl.ds(cc * LANES, LANES)
                    zr[b, r, sl] = zr[b, r, sl] * xr[b, r, sl]
                return c2

            lax.fori_loop(0, CHUNK, row_body, 0, unroll=False)

        start_gather(base, 0)

        def pair_body(p, carry):
            off0 = base + (2 * p) * CHUNK

            @pl.when(p > 0)
            def _():
                wait_write(1)

            start_gather(off0 + CHUNK, 1)
            wait_gather(0)
            compute(0)
            pltpu.async_copy(zr.at[0], out_hbm.at[pl.ds(off0, CHUNK)], osem[0])

            @pl.when(p < n_pairs - 1)
            def _():
                wait_write(0)
                start_gather(off0 + 2 * CHUNK, 0)

            wait_gather(1)
            compute(1)
            pltpu.async_copy(
                zr.at[1], out_hbm.at[pl.ds(off0 + CHUNK, CHUNK)], osem[1])
            return carry

        lax.fori_loop(0, n_pairs, pair_body, 0, unroll=False)
        wait_write(0)
        wait_write(1)

    return k


def kernel(z, x, edge_index):
    e = edge_index.shape[1]
    d = z.shape[1]
    src = edge_index[0].astype(jnp.int32)
    dst = edge_index[1].astype(jnp.int32)
    return _make_sc_kernel(e, d)(z, x, src, dst)
